# baseline (speedup 1.0000x reference)
import jax
import jax.numpy as jnp
from jax import lax
from jax.experimental import pallas as pl
from jax.experimental.pallas import tpu as pltpu
from jax.experimental.pallas import tpu_sc as plsc

N = 10000
D = 128
E = 160000
NC = 2
NS = 16
EPW = E // NS
BLK = 125
NBLK = EPW // BLK
CH = 16
NCH = NBLK // CH
NPAD = 10240
RPS = NPAD // NS
ZCH = 64
NZ = RPS // ZCH
DEGW = 128

_mesh = plsc.VectorSubcoreMesh(core_axis_name="c", subcore_axis_name="s")


def _agg_body(x_hbm, srcm_hbm, dstm_hbm,
              agg_hbm, sidx, didx, rows0, rows1, gsem0, gsem1, ssem0, ssem1,
              acc_sh):
    cid = lax.axis_index("c")
    sid = lax.axis_index("s")

    z16 = jnp.zeros((16,), jnp.float32)

    @pl.loop(0, ZCH)
    def _(i):
        @pl.loop(0, D, step=16)
        def _(j):
            rows0[i, pl.ds(j, 16)] = z16

    r0 = sid * RPS

    @pl.loop(0, NZ)
    def _(j):
        pltpu.sync_copy(rows0.at[pl.ds(0, ZCH)],
                        acc_sh.at[pl.ds(r0 + j * ZCH, ZCH)])

    plsc.subcore_barrier()

    def start(b, buf, gsem):
        pltpu.async_copy(x_hbm.at[sidx.at[b]], buf, gsem)

    def gwait(buf, gsem):
        pltpu.make_async_copy(x_hbm.at[sidx.at[0]], buf, gsem).wait()

    def scat(b, buf, ssem):
        pltpu.async_copy(buf, acc_sh.at[didx.at[b]], ssem, add=True)

    def swait(buf, ssem):
        pltpu.make_async_copy(buf, acc_sh.at[didx.at[0]], ssem).wait()

    @pl.loop(0, NCH)
    def _(c):
        pltpu.sync_copy(srcm_hbm.at[cid, sid, pl.ds(c * CH, CH)], sidx)
        pltpu.sync_copy(dstm_hbm.at[cid, sid, pl.ds(c * CH, CH)], didx)
        start(0, rows0, gsem0)
        start(1, rows1, gsem1)

        @pl.loop(0, CH - 2, step=2)
        def _(b):
            gwait(rows0, gsem0)
            scat(b, rows0, ssem0)
            gwait(rows1, gsem1)
            scat(b + 1, rows1, ssem1)
            swait(rows0, ssem0)
            start(b + 2, rows0, gsem0)
            swait(rows1, ssem1)
            start(b + 3, rows1, gsem1)

        gwait(rows0, gsem0)
        scat(CH - 2, rows0, ssem0)
        gwait(rows1, gsem1)
        scat(CH - 1, rows1, ssem1)
        swait(rows0, ssem0)
        swait(rows1, ssem1)

    plsc.subcore_barrier()

    @pl.loop(0, NZ)
    def _(j):
        rr = r0 + j * ZCH
        pltpu.sync_copy(acc_sh.at[pl.ds(rr, ZCH)],
                        agg_hbm.at[cid, pl.ds(rr, ZCH)])


_agg = pl.kernel(
    _agg_body,
    out_type=jax.ShapeDtypeStruct((NC, NPAD, D), jnp.float32),
    mesh=_mesh,
    scratch_types=[
        pltpu.VMEM((CH, BLK), jnp.int32),
        pltpu.VMEM((CH, BLK), jnp.int32),
        pltpu.VMEM((BLK, D), jnp.float32),
        pltpu.VMEM((BLK, D), jnp.float32),
        pltpu.SemaphoreType.DMA,
        pltpu.SemaphoreType.DMA,
        pltpu.SemaphoreType.DMA,
        pltpu.SemaphoreType.DMA,
        pltpu.VMEM_SHARED((NPAD, D), jnp.float32),
    ],
)


def _deg_body(dstm_hbm, deg_hbm, didx, ones, dsem, deg_sh):
    cid = lax.axis_index("c")
    sid = lax.axis_index("s")

    z16 = jnp.zeros((16,), jnp.float32)
    o16 = jnp.full((16,), 1.0, jnp.float32)

    @pl.loop(0, ZCH)
    def _(i):
        @pl.loop(0, DEGW, step=16)
        def _(j):
            ones[i, pl.ds(j, 16)] = z16

    r0 = sid * RPS

    @pl.loop(0, NZ)
    def _(j):
        pltpu.sync_copy(ones.at[pl.ds(0, ZCH)],
                        deg_sh.at[pl.ds(r0 + j * ZCH, ZCH)])

    plsc.subcore_barrier()

    @pl.loop(0, BLK)
    def _(i):
        @pl.loop(0, DEGW, step=16)
        def _(j):
            ones[i, pl.ds(j, 16)] = o16

    pltpu.sync_copy(dstm_hbm.at[cid, sid], didx)

    @pl.loop(0, NBLK)
    def _(b):
        pltpu.async_copy(ones, deg_sh.at[didx.at[b]], dsem, add=True)

    @pl.loop(0, NBLK)
    def _(b):
        pltpu.make_async_copy(ones, deg_sh.at[didx.at[0]], dsem).wait()

    plsc.subcore_barrier()

    @pl.loop(0, NZ)
    def _(j):
        rr = r0 + j * ZCH
        pltpu.sync_copy(deg_sh.at[pl.ds(rr, ZCH)],
                        deg_hbm.at[cid, pl.ds(rr, ZCH)])


_deg = pl.kernel(
    _deg_body,
    out_type=jax.ShapeDtypeStruct((NC, NPAD, DEGW), jnp.float32),
    mesh=_mesh,
    scratch_types=[
        pltpu.VMEM((NBLK, BLK), jnp.int32),
        pltpu.VMEM((BLK, DEGW), jnp.float32),
        pltpu.SemaphoreType.DMA,
        pltpu.VMEM_SHARED((NPAD, DEGW), jnp.float32),
    ],
)

BN = 1000


def _make_dense(relu: bool):
    def body(x_r, agg_r, deg_r, ws0_r, ws1_r, wn0_r, wn1_r, b0_r, b1_r, o_r):
        ws = ws0_r[...] + ws1_r[...]
        b = b0_r[...] + b1_r[...]
        inv0 = 1.0 / jnp.maximum(deg_r[0, :, 0:1], 1.0)
        inv1 = 1.0 / jnp.maximum(deg_r[1, :, 0:1], 1.0)
        h = (jnp.dot(x_r[...], ws, preferred_element_type=jnp.float32)
             + jnp.dot(agg_r[0] * inv0, wn0_r[...],
                       preferred_element_type=jnp.float32)
             + jnp.dot(agg_r[1] * inv1, wn1_r[...],
                       preferred_element_type=jnp.float32)
             + b)
        o_r[...] = jnp.maximum(h, 0.0) if relu else h

    full = lambda i: (0, 0)
    return pl.pallas_call(
        body,
        grid=(N // BN,),
        in_specs=[
            pl.BlockSpec((BN, D), lambda i: (i, 0)),
            pl.BlockSpec((NC, BN, D), lambda i: (0, i, 0)),
            pl.BlockSpec((NC, BN, DEGW), lambda i: (0, i, 0)),
            pl.BlockSpec((D, D), full),
            pl.BlockSpec((D, D), full),
            pl.BlockSpec((D, D), full),
            pl.BlockSpec((D, D), full),
            pl.BlockSpec((1, D), full),
            pl.BlockSpec((1, D), full),
        ],
        out_specs=pl.BlockSpec((BN, D), lambda i: (i, 0)),
        out_shape=jax.ShapeDtypeStruct((N, D), jnp.float32),
    )


_dense_relu = _make_dense(True)
_dense_lin = _make_dense(False)


def kernel(x, edge_index_r0, edge_index_r1,
           Ws0_r0, Wn0_r0, b0_r0, Ws0_r1, Wn0_r1, b0_r1,
           Ws1_r0, Wn1_r0, b1_r0, Ws1_r1, Wn1_r1, b1_r1):
    src = jnp.stack([edge_index_r0[0], edge_index_r1[0]]).astype(jnp.int32)
    dst = jnp.stack([edge_index_r0[1], edge_index_r1[1]]).astype(jnp.int32)
    srcm = src.reshape(NC, NS, NBLK, BLK)
    dstm = dst.reshape(NC, NS, NBLK, BLK)

    deg = _deg(dstm)
    agg0 = _agg(x, srcm, dstm)
    h = _dense_relu(x, agg0, deg, Ws0_r0, Ws0_r1, Wn0_r0, Wn0_r1,
                    b0_r0.reshape(1, D), b0_r1.reshape(1, D))
    agg1 = _agg(h, srcm, dstm)
    out = _dense_lin(h, agg1, deg, Ws1_r0, Ws1_r1, Wn1_r0, Wn1_r1,
                     b1_r0.reshape(1, D), b1_r1.reshape(1, D))
    return out


# speedup vs baseline: 1.1828x; 1.1828x over previous
import jax
import jax.numpy as jnp
from jax import lax
from jax.experimental import pallas as pl
from jax.experimental.pallas import tpu as pltpu
from jax.experimental.pallas import tpu_sc as plsc

N = 10000
D = 128
E = 160000
NC = 2
NS = 16
EPW = E // NS
BLK = 125
NBLK = EPW // BLK
CH = 16
NCH = NBLK // CH
NPAD = 10240
RPS = NPAD // NS
ZCH = 64
NZ = RPS // ZCH
DEGW = 128

_mesh = plsc.VectorSubcoreMesh(core_axis_name="c", subcore_axis_name="s")


def _agg_body(x_hbm, srcm_hbm, dstm_hbm,
              agg_hbm, sidx, didx, rows0, rows1, gsem0, gsem1, acc_sh):
    cid = lax.axis_index("c")
    sid = lax.axis_index("s")

    z16 = jnp.zeros((16,), jnp.float32)

    @pl.loop(0, ZCH)
    def _(i):
        @pl.loop(0, D, step=16)
        def _(j):
            rows0[i, pl.ds(j, 16)] = z16

    r0 = sid * RPS

    @pl.loop(0, NZ)
    def _(j):
        pltpu.sync_copy(rows0.at[pl.ds(0, ZCH)],
                        acc_sh.at[pl.ds(r0 + j * ZCH, ZCH)])

    plsc.subcore_barrier()

    def start(b, buf, gsem):
        pltpu.async_copy(x_hbm.at[sidx.at[b]], buf, gsem)

    def gwait(buf, gsem):
        pltpu.make_async_copy(x_hbm.at[sidx.at[0]], buf, gsem).wait()

    @pl.loop(0, NCH)
    def _(c):
        pltpu.sync_copy(srcm_hbm.at[cid, sid, pl.ds(c * CH, CH)], sidx)
        pltpu.sync_copy(dstm_hbm.at[cid, sid, pl.ds(c * CH, CH)], didx)
        start(0, rows0, gsem0)
        start(1, rows1, gsem1)

        @pl.loop(0, CH - 2, step=2)
        def _(b):
            gwait(rows0, gsem0)
            pltpu.sync_copy(rows0, acc_sh.at[didx.at[b]], add=True)
            start(b + 2, rows0, gsem0)
            gwait(rows1, gsem1)
            pltpu.sync_copy(rows1, acc_sh.at[didx.at[b + 1]], add=True)
            start(b + 3, rows1, gsem1)

        gwait(rows0, gsem0)
        pltpu.sync_copy(rows0, acc_sh.at[didx.at[CH - 2]], add=True)
        gwait(rows1, gsem1)
        pltpu.sync_copy(rows1, acc_sh.at[didx.at[CH - 1]], add=True)

    plsc.subcore_barrier()

    @pl.loop(0, NZ)
    def _(j):
        rr = r0 + j * ZCH
        pltpu.sync_copy(acc_sh.at[pl.ds(rr, ZCH)],
                        agg_hbm.at[cid, pl.ds(rr, ZCH)])


_agg = pl.kernel(
    _agg_body,
    out_type=jax.ShapeDtypeStruct((NC, NPAD, D), jnp.float32),
    mesh=_mesh,
    scratch_types=[
        pltpu.VMEM((CH, BLK), jnp.int32),
        pltpu.VMEM((CH, BLK), jnp.int32),
        pltpu.VMEM((BLK, D), jnp.float32),
        pltpu.VMEM((BLK, D), jnp.float32),
        pltpu.SemaphoreType.DMA,
        pltpu.SemaphoreType.DMA,
        pltpu.VMEM_SHARED((NPAD, D), jnp.float32),
    ],
)


def _deg_body(dstm_hbm, deg_hbm, didx, ones, dsem, deg_sh):
    cid = lax.axis_index("c")
    sid = lax.axis_index("s")

    z16 = jnp.zeros((16,), jnp.float32)
    o16 = jnp.full((16,), 1.0, jnp.float32)

    @pl.loop(0, ZCH)
    def _(i):
        @pl.loop(0, DEGW, step=16)
        def _(j):
            ones[i, pl.ds(j, 16)] = z16

    r0 = sid * RPS

    @pl.loop(0, NZ)
    def _(j):
        pltpu.sync_copy(ones.at[pl.ds(0, ZCH)],
                        deg_sh.at[pl.ds(r0 + j * ZCH, ZCH)])

    plsc.subcore_barrier()

    @pl.loop(0, BLK)
    def _(i):
        @pl.loop(0, DEGW, step=16)
        def _(j):
            ones[i, pl.ds(j, 16)] = o16

    pltpu.sync_copy(dstm_hbm.at[cid, sid], didx)

    @pl.loop(0, NBLK)
    def _(b):
        pltpu.async_copy(ones, deg_sh.at[didx.at[b]], dsem, add=True)

    @pl.loop(0, NBLK)
    def _(b):
        pltpu.make_async_copy(ones, deg_sh.at[didx.at[0]], dsem).wait()

    plsc.subcore_barrier()

    @pl.loop(0, NZ)
    def _(j):
        rr = r0 + j * ZCH
        pltpu.sync_copy(deg_sh.at[pl.ds(rr, ZCH)],
                        deg_hbm.at[cid, pl.ds(rr, ZCH)])


_deg = pl.kernel(
    _deg_body,
    out_type=jax.ShapeDtypeStruct((NC, NPAD, DEGW), jnp.float32),
    mesh=_mesh,
    scratch_types=[
        pltpu.VMEM((NBLK, BLK), jnp.int32),
        pltpu.VMEM((BLK, DEGW), jnp.float32),
        pltpu.SemaphoreType.DMA,
        pltpu.VMEM_SHARED((NPAD, DEGW), jnp.float32),
    ],
)

BN = 1000


def _make_dense(relu: bool):
    def body(x_r, agg_r, deg_r, ws0_r, ws1_r, wn0_r, wn1_r, b0_r, b1_r, o_r):
        ws = ws0_r[...] + ws1_r[...]
        b = b0_r[...] + b1_r[...]
        inv0 = 1.0 / jnp.maximum(deg_r[0, :, 0:1], 1.0)
        inv1 = 1.0 / jnp.maximum(deg_r[1, :, 0:1], 1.0)
        h = (jnp.dot(x_r[...], ws, preferred_element_type=jnp.float32)
             + jnp.dot(agg_r[0] * inv0, wn0_r[...],
                       preferred_element_type=jnp.float32)
             + jnp.dot(agg_r[1] * inv1, wn1_r[...],
                       preferred_element_type=jnp.float32)
             + b)
        o_r[...] = jnp.maximum(h, 0.0) if relu else h

    full = lambda i: (0, 0)
    return pl.pallas_call(
        body,
        grid=(N // BN,),
        in_specs=[
            pl.BlockSpec((BN, D), lambda i: (i, 0)),
            pl.BlockSpec((NC, BN, D), lambda i: (0, i, 0)),
            pl.BlockSpec((NC, BN, DEGW), lambda i: (0, i, 0)),
            pl.BlockSpec((D, D), full),
            pl.BlockSpec((D, D), full),
            pl.BlockSpec((D, D), full),
            pl.BlockSpec((D, D), full),
            pl.BlockSpec((1, D), full),
            pl.BlockSpec((1, D), full),
        ],
        out_specs=pl.BlockSpec((BN, D), lambda i: (i, 0)),
        out_shape=jax.ShapeDtypeStruct((N, D), jnp.float32),
    )


_dense_relu = _make_dense(True)
_dense_lin = _make_dense(False)


def kernel(x, edge_index_r0, edge_index_r1,
           Ws0_r0, Wn0_r0, b0_r0, Ws0_r1, Wn0_r1, b0_r1,
           Ws1_r0, Wn1_r0, b1_r0, Ws1_r1, Wn1_r1, b1_r1):
    src = jnp.stack([edge_index_r0[0], edge_index_r1[0]]).astype(jnp.int32)
    dst = jnp.stack([edge_index_r0[1], edge_index_r1[1]]).astype(jnp.int32)
    srcm = src.reshape(NC, NS, NBLK, BLK)
    dstm = dst.reshape(NC, NS, NBLK, BLK)

    deg = _deg(dstm)
    agg0 = _agg(x, srcm, dstm)
    h = _dense_relu(x, agg0, deg, Ws0_r0, Ws0_r1, Wn0_r0, Wn0_r1,
                    b0_r0.reshape(1, D), b0_r1.reshape(1, D))
    agg1 = _agg(h, srcm, dstm)
    out = _dense_lin(h, agg1, deg, Ws1_r0, Ws1_r1, Wn1_r0, Wn1_r1,
                     b1_r0.reshape(1, D), b1_r1.reshape(1, D))
    return out


# speedup vs baseline: 1.2497x; 1.0566x over previous
import jax
import jax.numpy as jnp
from jax import lax
from jax.experimental import pallas as pl
from jax.experimental.pallas import tpu as pltpu
from jax.experimental.pallas import tpu_sc as plsc

N = 10000
D = 128
E = 160000
NC = 2
NS = 16
EPW = E // NS
BLK = 125
NBLK = EPW // BLK
CH = 40
NCH = NBLK // CH
NPAD = 10240
RPS = NPAD // NS
ZCH = 64
NZ = RPS // ZCH
DEGW = 128

_mesh = plsc.VectorSubcoreMesh(core_axis_name="c", subcore_axis_name="s")


def _agg_body(x_hbm, srcm_hbm, dstm_hbm,
              agg_hbm, sidx, didx, rows0, rows1, gsem0, gsem1, acc_sh):
    cid = lax.axis_index("c")
    sid = lax.axis_index("s")

    z16 = jnp.zeros((16,), jnp.float32)

    @pl.loop(0, ZCH)
    def _(i):
        @pl.loop(0, D, step=16)
        def _(j):
            rows0[i, pl.ds(j, 16)] = z16

    r0 = sid * RPS

    @pl.loop(0, NZ)
    def _(j):
        pltpu.async_copy(rows0.at[pl.ds(0, ZCH)],
                         acc_sh.at[pl.ds(r0 + j * ZCH, ZCH)], gsem0)

    @pl.loop(0, NZ)
    def _(j):
        pltpu.make_async_copy(rows0.at[pl.ds(0, ZCH)],
                              acc_sh.at[pl.ds(r0, ZCH)], gsem0).wait()

    plsc.subcore_barrier()

    def start(b, buf, gsem):
        pltpu.async_copy(x_hbm.at[sidx.at[b]], buf, gsem)

    def gwait(buf, gsem):
        pltpu.make_async_copy(x_hbm.at[sidx.at[0]], buf, gsem).wait()

    @pl.loop(0, NCH)
    def _(c):
        pltpu.sync_copy(srcm_hbm.at[cid, sid, pl.ds(c * CH, CH)], sidx)
        pltpu.sync_copy(dstm_hbm.at[cid, sid, pl.ds(c * CH, CH)], didx)
        start(0, rows0, gsem0)
        start(1, rows1, gsem1)

        @pl.loop(0, CH - 2, step=2)
        def _(b):
            gwait(rows0, gsem0)
            pltpu.sync_copy(rows0, acc_sh.at[didx.at[b]], add=True)
            start(b + 2, rows0, gsem0)
            gwait(rows1, gsem1)
            pltpu.sync_copy(rows1, acc_sh.at[didx.at[b + 1]], add=True)
            start(b + 3, rows1, gsem1)

        gwait(rows0, gsem0)
        pltpu.sync_copy(rows0, acc_sh.at[didx.at[CH - 2]], add=True)
        gwait(rows1, gsem1)
        pltpu.sync_copy(rows1, acc_sh.at[didx.at[CH - 1]], add=True)

    plsc.subcore_barrier()

    @pl.loop(0, NZ)
    def _(j):
        rr = r0 + j * ZCH
        pltpu.async_copy(acc_sh.at[pl.ds(rr, ZCH)],
                         agg_hbm.at[cid, pl.ds(rr, ZCH)], gsem0)

    @pl.loop(0, NZ)
    def _(j):
        pltpu.make_async_copy(acc_sh.at[pl.ds(r0, ZCH)],
                              agg_hbm.at[cid, pl.ds(r0, ZCH)], gsem0).wait()


_agg = pl.kernel(
    _agg_body,
    out_type=jax.ShapeDtypeStruct((NC, NPAD, D), jnp.float32),
    mesh=_mesh,
    scratch_types=[
        pltpu.VMEM((CH, BLK), jnp.int32),
        pltpu.VMEM((CH, BLK), jnp.int32),
        pltpu.VMEM((BLK, D), jnp.float32),
        pltpu.VMEM((BLK, D), jnp.float32),
        pltpu.SemaphoreType.DMA,
        pltpu.SemaphoreType.DMA,
        pltpu.VMEM_SHARED((NPAD, D), jnp.float32),
    ],
)


def _deg_body(dstm_hbm, deg_hbm, didx, ones, dsem, deg_sh):
    cid = lax.axis_index("c")
    sid = lax.axis_index("s")

    z16 = jnp.zeros((16,), jnp.float32)
    o16 = jnp.full((16,), 1.0, jnp.float32)

    @pl.loop(0, ZCH)
    def _(i):
        @pl.loop(0, DEGW, step=16)
        def _(j):
            ones[i, pl.ds(j, 16)] = z16

    r0 = sid * RPS

    @pl.loop(0, NZ)
    def _(j):
        pltpu.async_copy(ones.at[pl.ds(0, ZCH)],
                         deg_sh.at[pl.ds(r0 + j * ZCH, ZCH)], dsem)

    @pl.loop(0, NZ)
    def _(j):
        pltpu.make_async_copy(ones.at[pl.ds(0, ZCH)],
                              deg_sh.at[pl.ds(r0, ZCH)], dsem).wait()

    plsc.subcore_barrier()

    @pl.loop(0, BLK)
    def _(i):
        @pl.loop(0, DEGW, step=16)
        def _(j):
            ones[i, pl.ds(j, 16)] = o16

    pltpu.sync_copy(dstm_hbm.at[cid, sid], didx)

    @pl.loop(0, NBLK)
    def _(b):
        pltpu.async_copy(ones, deg_sh.at[didx.at[b]], dsem, add=True)

    @pl.loop(0, NBLK)
    def _(b):
        pltpu.make_async_copy(ones, deg_sh.at[didx.at[0]], dsem).wait()

    plsc.subcore_barrier()

    @pl.loop(0, NZ)
    def _(j):
        rr = r0 + j * ZCH
        pltpu.async_copy(deg_sh.at[pl.ds(rr, ZCH)],
                         deg_hbm.at[cid, pl.ds(rr, ZCH)], dsem)

    @pl.loop(0, NZ)
    def _(j):
        pltpu.make_async_copy(deg_sh.at[pl.ds(r0, ZCH)],
                              deg_hbm.at[cid, pl.ds(r0, ZCH)], dsem).wait()


_deg = pl.kernel(
    _deg_body,
    out_type=jax.ShapeDtypeStruct((NC, NPAD, DEGW), jnp.float32),
    mesh=_mesh,
    scratch_types=[
        pltpu.VMEM((NBLK, BLK), jnp.int32),
        pltpu.VMEM((BLK, DEGW), jnp.float32),
        pltpu.SemaphoreType.DMA,
        pltpu.VMEM_SHARED((NPAD, DEGW), jnp.float32),
    ],
)

BN = 1000


def _make_dense(relu: bool):
    def body(x_r, agg_r, deg_r, ws0_r, ws1_r, wn0_r, wn1_r, b0_r, b1_r, o_r):
        ws = ws0_r[...] + ws1_r[...]
        b = b0_r[...] + b1_r[...]
        inv0 = 1.0 / jnp.maximum(deg_r[0, :, 0:1], 1.0)
        inv1 = 1.0 / jnp.maximum(deg_r[1, :, 0:1], 1.0)
        h = (jnp.dot(x_r[...], ws, preferred_element_type=jnp.float32)
             + jnp.dot(agg_r[0] * inv0, wn0_r[...],
                       preferred_element_type=jnp.float32)
             + jnp.dot(agg_r[1] * inv1, wn1_r[...],
                       preferred_element_type=jnp.float32)
             + b)
        o_r[...] = jnp.maximum(h, 0.0) if relu else h

    full = lambda i: (0, 0)
    return pl.pallas_call(
        body,
        grid=(N // BN,),
        in_specs=[
            pl.BlockSpec((BN, D), lambda i: (i, 0)),
            pl.BlockSpec((NC, BN, D), lambda i: (0, i, 0)),
            pl.BlockSpec((NC, BN, DEGW), lambda i: (0, i, 0)),
            pl.BlockSpec((D, D), full),
            pl.BlockSpec((D, D), full),
            pl.BlockSpec((D, D), full),
            pl.BlockSpec((D, D), full),
            pl.BlockSpec((1, D), full),
            pl.BlockSpec((1, D), full),
        ],
        out_specs=pl.BlockSpec((BN, D), lambda i: (i, 0)),
        out_shape=jax.ShapeDtypeStruct((N, D), jnp.float32),
    )


_dense_relu = _make_dense(True)
_dense_lin = _make_dense(False)


def kernel(x, edge_index_r0, edge_index_r1,
           Ws0_r0, Wn0_r0, b0_r0, Ws0_r1, Wn0_r1, b0_r1,
           Ws1_r0, Wn1_r0, b1_r0, Ws1_r1, Wn1_r1, b1_r1):
    src = jnp.stack([edge_index_r0[0], edge_index_r1[0]]).astype(jnp.int32)
    dst = jnp.stack([edge_index_r0[1], edge_index_r1[1]]).astype(jnp.int32)
    srcm = src.reshape(NC, NS, NBLK, BLK)
    dstm = dst.reshape(NC, NS, NBLK, BLK)

    deg = _deg(dstm)
    agg0 = _agg(x, srcm, dstm)
    h = _dense_relu(x, agg0, deg, Ws0_r0, Ws0_r1, Wn0_r0, Wn0_r1,
                    b0_r0.reshape(1, D), b0_r1.reshape(1, D))
    agg1 = _agg(h, srcm, dstm)
    out = _dense_lin(h, agg1, deg, Ws1_r0, Ws1_r1, Wn1_r0, Wn1_r1,
                     b1_r0.reshape(1, D), b1_r1.reshape(1, D))
    return out
